# baseline (speedup 1.0000x reference)
import numpy as np
import jax
import jax.numpy as jnp
from jax.experimental import pallas as pl

_FEAT_STRIDE = 8.0
_PRE = 1000
_POST = 300
_THR = 0.7
_PAD = 1024
_NA = 9


def _anchor_table():
    base_size = 16
    size = float(base_size * base_size)
    ctr = (base_size - 1) / 2.0
    rows = []
    for r in (0.5, 1.0, 2.0):
        ws0 = np.round(np.sqrt(size / r))
        hs0 = np.round(ws0 * r)
        for s in (4.0, 8.0, 16.0):
            w = ws0 * s
            h = hs0 * s
            d = base_size * s
            rows.append([ctr - 0.5 * (w - 1), ctr - 0.5 * (h - 1),
                         ctr + 0.5 * (w - 1), ctr + 0.5 * (h - 1),
                         ctr - 0.5 * (d - 1), ctr + 0.5 * (d - 1)])
    return np.array(rows, dtype=np.float32)


_ANCH = _anchor_table()


def _select_body(sc_ref, sel_ref):
    x = sc_ref[0]
    rows = x.shape[0]
    n = rows * 1024
    ii = jax.lax.bitcast_convert_type(x, jnp.int32)
    mapped = jnp.where(ii < 0, ii ^ jnp.int32(0x7FFFFFFF), ii)
    k = jnp.int32(_PRE)

    def cnt_ge(t):
        return jnp.sum((mapped >= t).astype(jnp.int32))

    big = cnt_ge(jnp.int32(0)) >= k
    lo = jnp.where(big, jnp.int32(0), jnp.int32(-2147483648))
    hi = jnp.where(big, jnp.int32(2147483647), jnp.int32(-1))

    def val_step(_, carry):
        lo, hi = carry
        mid = lo + (hi - lo) // 2
        go_up = cnt_ge(mid + 1) >= k
        return jnp.where(go_up, mid + 1, lo), jnp.where(go_up, hi, mid)

    lo, hi = jax.lax.fori_loop(0, 31, val_step, (lo, hi))
    thresh = lo
    extra = k - cnt_ge(thresh + 1)

    ri = jax.lax.broadcasted_iota(jnp.int32, (rows, 1024), 0)
    ci = jax.lax.broadcasted_iota(jnp.int32, (rows, 1024), 1)
    flat = ri * 1024 + ci
    is_tie = mapped == thresh

    def idx_step(_, carry):
        lo, hi = carry
        mid = lo + (hi - lo) // 2
        cnt = jnp.sum((is_tie & (flat <= mid)).astype(jnp.int32))
        enough = cnt >= extra
        return jnp.where(enough, lo, mid + 1), jnp.where(enough, mid, hi)

    tlo, _ = jax.lax.fori_loop(0, 19, idx_step, (jnp.int32(0), jnp.int32(n - 1)))
    sel = (mapped > thresh) | (is_tie & (flat <= tlo))
    sel_ref[0] = sel.astype(jnp.int32)


def _nms_body(anch_ref, delt_ref, sc_ref, info_ref, boxes_ref, masked_ref):
    a = anch_ref[0]
    dl = delt_ref[0]
    sc = sc_ref[0][0]
    h_im = info_ref[0, 0, 0]
    w_im = info_ref[0, 0, 1]
    d_im = info_ref[0, 0, 2]

    widths = a[2] - a[0] + 1.0
    heights = a[3] - a[1] + 1.0
    depths = a[5] - a[4] + 1.0
    ctr_x = a[0] + 0.5 * widths
    ctr_y = a[1] + 0.5 * heights
    ctr_z = a[4] + 0.5 * depths
    pcx = dl[0] * widths + ctr_x
    pcy = dl[1] * heights + ctr_y
    pcz = dl[2] * depths + ctr_z
    pw = jnp.exp(dl[3]) * widths
    ph = jnp.exp(dl[4]) * heights
    pd = jnp.exp(dl[5]) * depths

    x1 = jnp.clip(pcx - 0.5 * pw, 0.0, w_im - 1.0)
    y1 = jnp.clip(pcy - 0.5 * ph, 0.0, h_im - 1.0)
    x2 = jnp.clip(pcx + 0.5 * pw, 0.0, w_im - 1.0)
    y2 = jnp.clip(pcy + 0.5 * ph, 0.0, h_im - 1.0)
    z1 = jnp.clip(pcz - 0.5 * pd, 0.0, d_im - 1.0)
    z2 = jnp.clip(pcz + 0.5 * pd, 0.0, d_im - 1.0)

    area = (x2 - x1) * (y2 - y1) * (z2 - z1)
    iw = jnp.maximum(
        jnp.minimum(x2[:, None], x2[None, :]) - jnp.maximum(x1[:, None], x1[None, :]), 0.0)
    ih = jnp.maximum(
        jnp.minimum(y2[:, None], y2[None, :]) - jnp.maximum(y1[:, None], y1[None, :]), 0.0)
    idp = jnp.maximum(
        jnp.minimum(z2[:, None], z2[None, :]) - jnp.maximum(z1[:, None], z1[None, :]), 0.0)
    inter = iw * ih * idp
    ua = jnp.maximum(area[:, None] + area[None, :] - inter, 1e-8)
    iou = inter / ua
    ri = jax.lax.broadcasted_iota(jnp.int32, (_PAD, _PAD), 0)
    ci = jax.lax.broadcasted_iota(jnp.int32, (_PAD, _PAD), 1)
    iou = jnp.where(ci > ri, iou, 0.0)

    c = iou
    for _ in range(4):
        keep_f = (c.max(axis=0) < _THR).astype(jnp.float32)
        c = iou * keep_f[:, None]
    keep = c.max(axis=0) < _THR

    eye = (ri == ci).astype(jnp.float32)
    wm = (c * (c > _THR).astype(jnp.float32) + eye) * sc[None, :]
    wsum = wm.sum(axis=1)
    boxes_ref[0, 0] = (wm * x1[None, :]).sum(axis=1) / wsum
    boxes_ref[0, 1] = (wm * y1[None, :]).sum(axis=1) / wsum
    boxes_ref[0, 2] = (wm * x2[None, :]).sum(axis=1) / wsum
    boxes_ref[0, 3] = (wm * y2[None, :]).sum(axis=1) / wsum
    boxes_ref[0, 4] = (wm * z1[None, :]).sum(axis=1) / wsum
    boxes_ref[0, 5] = (wm * z2[None, :]).sum(axis=1) / wsum
    boxes_ref[0, 6] = jnp.zeros((_PAD,), jnp.float32)
    boxes_ref[0, 7] = jnp.zeros((_PAD,), jnp.float32)
    masked_ref[0, 0] = jnp.where(keep, sc, -1e9)


def kernel(scores, bbox_deltas, im_info):
    B, _, H, W, D = scores.shape

    sc = jnp.transpose(scores[:, _NA:, :, :, :], (0, 2, 3, 4, 1)).reshape(B, -1)
    n = sc.shape[1]
    rows = n // 1024

    sel = pl.pallas_call(
        _select_body,
        grid=(B,),
        in_specs=[pl.BlockSpec((1, rows, 1024), lambda b: (b, 0, 0))],
        out_specs=pl.BlockSpec((1, rows, 1024), lambda b: (b, 0, 0)),
        out_shape=jax.ShapeDtypeStruct((B, rows, 1024), jnp.int32),
    )(sc.reshape(B, rows, 1024)).reshape(B, n)

    pos = jnp.cumsum(sel, axis=1) - 1
    flat_ids = jnp.broadcast_to(jnp.arange(n, dtype=jnp.int32)[None], (B, n))
    pos = jnp.where(sel.astype(bool), pos, _PRE)

    def _compact(p, f):
        return jnp.zeros((_PRE,), jnp.int32).at[p].set(f, mode="drop")

    idx_asc = jax.vmap(_compact)(pos, flat_ids)
    vals_asc = jnp.take_along_axis(sc, idx_asc, axis=1)
    _, idx, vals = jax.lax.sort((-vals_asc, idx_asc, vals_asc),
                                dimension=1, is_stable=True, num_keys=1)

    a_i = idx % _NA
    sp = idx // _NA
    d_i = sp % D
    w_i = (sp // D) % W
    h_i = sp // (D * W)

    bd = bbox_deltas.reshape(B, -1)
    delt_rows = []
    for coord in range(6):
        ch = 6 * a_i + coord
        fi = ((ch * H + h_i) * W + w_i) * D + d_i
        delt_rows.append(jnp.take_along_axis(bd, fi, axis=1))
    delt = jnp.stack(delt_rows, axis=1)

    base = jnp.asarray(_ANCH)[a_i]
    shift = jnp.stack([w_i, h_i, w_i, h_i, d_i, d_i],
                      axis=-1).astype(jnp.float32) * _FEAT_STRIDE
    anch = jnp.transpose(base + shift, (0, 2, 1))

    anch8 = jnp.zeros((B, 8, _PAD), jnp.float32).at[:, :6, :_PRE].set(anch)
    delt8 = jnp.zeros((B, 8, _PAD), jnp.float32).at[:, :6, :_PRE].set(delt)
    sc_in = jnp.zeros((B, 1, _PAD), jnp.float32).at[:, 0, :_PRE].set(vals)
    info = im_info.reshape(B, 1, 3)

    boxes_out, masked_out = pl.pallas_call(
        _nms_body,
        grid=(B,),
        in_specs=[
            pl.BlockSpec((1, 8, _PAD), lambda b: (b, 0, 0)),
            pl.BlockSpec((1, 8, _PAD), lambda b: (b, 0, 0)),
            pl.BlockSpec((1, 1, _PAD), lambda b: (b, 0, 0)),
            pl.BlockSpec((1, 1, 3), lambda b: (b, 0, 0)),
        ],
        out_specs=[
            pl.BlockSpec((1, 8, _PAD), lambda b: (b, 0, 0)),
            pl.BlockSpec((1, 1, _PAD), lambda b: (b, 0, 0)),
        ],
        out_shape=[
            jax.ShapeDtypeStruct((B, 8, _PAD), jnp.float32),
            jax.ShapeDtypeStruct((B, 1, _PAD), jnp.float32),
        ],
    )(anch8, delt8, sc_in, info)

    masked = masked_out[:, 0, :_PRE]
    _, kidx = jax.lax.top_k(masked, _POST)
    nb = boxes_out[:, :6, :_PRE]
    sel = jnp.take_along_axis(nb, jnp.broadcast_to(kidx[:, None, :], (B, 6, _POST)), axis=2)
    sel = jnp.transpose(sel, (0, 2, 1))
    bid = jnp.broadcast_to(
        jnp.arange(B, dtype=jnp.float32)[:, None, None], (B, _POST, 1))
    return jnp.concatenate([bid, sel], axis=2)
